# Initial kernel scaffold; baseline (speedup 1.0000x reference)
import jax
import jax.numpy as jnp
from jax.experimental import pallas as pl


def kernel(x, edge_index, W1, b1, W3, b3, W4, b4, W2, b2):
    raise NotImplementedError("write your pallas kernel here")


# speedup vs baseline: 11.6753x; 11.6753x over previous
import functools

import jax
import jax.numpy as jnp
from jax import lax
from jax.experimental import pallas as pl
from jax.experimental.pallas import tpu as pltpu
from jax.experimental.pallas import tpu_sc as plsc

_N = 10000
_E = 320000
_NC = 2
_NS = 16
_NW = _NC * _NS
_EPT = _E // _NW
_CHUNK = 80
_STEPS = _EPT // _CHUNK
_RPT = 640
_RLAST = _N - _RPT
_ZR = 128
_DEGW = 16

_BN = 1000


def _zero_vmem(ref, rows, w):
    zero = jnp.zeros((16,), jnp.float32)

    def body(i, _):
        def inner(j, __):
            ref[i, pl.ds(j * 16, 16)] = zero
            return 0

        return lax.fori_loop(0, w // 16, inner, 0)

    lax.fori_loop(0, rows, body, 0)


def _fill_vmem(ref, rows, w, value):
    val = jnp.full((16,), value, jnp.float32)

    def body(i, _):
        def inner(j, __):
            ref[i, pl.ds(j * 16, 16)] = val
            return 0

        return lax.fori_loop(0, w // 16, inner, 0)

    lax.fori_loop(0, rows, body, 0)


_sc_mesh = plsc.VectorSubcoreMesh(core_axis_name="c", subcore_axis_name="s")


@functools.partial(
    pl.kernel,
    out_type=jax.ShapeDtypeStruct((_NC, _N, _DEGW), jnp.float32),
    mesh=_sc_mesh,
    scratch_types=[
        pltpu.VMEM((_CHUNK,), jnp.int32),
        pltpu.VMEM((_CHUNK, _DEGW), jnp.float32),
        pltpu.VMEM((_ZR, _DEGW), jnp.float32),
        pltpu.VMEM_SHARED((_N, _DEGW), jnp.float32),
    ],
    compiler_params=pltpu.CompilerParams(use_tc_tiling_on_sc=False),
)
def _sc_degree(dst_hbm, out_hbm, dstv, ones, zblk, acc):
    c = lax.axis_index("c")
    s = lax.axis_index("s")
    row0 = jnp.minimum(s * _RPT, _RLAST)
    _zero_vmem(zblk, _ZR, _DEGW)
    _fill_vmem(ones, _CHUNK, _DEGW, 1.0)
    for k in range(_RPT // _ZR):
        pltpu.sync_copy(zblk, acc.at[pl.ds(row0 + k * _ZR, _ZR)])
    plsc.subcore_barrier()
    base = (c * _NS + s) * _EPT

    def step(i, _):
        pltpu.sync_copy(dst_hbm.at[pl.ds(base + i * _CHUNK, _CHUNK)], dstv)
        pltpu.sync_copy(ones, acc.at[dstv], add=True)
        return 0

    lax.fori_loop(0, _STEPS, step, 0)
    plsc.subcore_barrier()
    pltpu.sync_copy(acc.at[pl.ds(row0, _RPT)], out_hbm.at[c, pl.ds(row0, _RPT)])


def _make_sc_agg(w):
    @functools.partial(
        pl.kernel,
        out_type=jax.ShapeDtypeStruct((_NC, _N, w), jnp.float32),
        mesh=_sc_mesh,
        scratch_types=[
            pltpu.VMEM((_CHUNK,), jnp.int32),
            pltpu.VMEM((_CHUNK,), jnp.int32),
            pltpu.VMEM((_CHUNK, w), jnp.float32),
            pltpu.VMEM((_ZR, w), jnp.float32),
            pltpu.VMEM_SHARED((_N, w), jnp.float32),
            pltpu.SemaphoreType.DMA,
        ],
        compiler_params=pltpu.CompilerParams(use_tc_tiling_on_sc=False),
    )
    def agg(q_hbm, src_hbm, dst_hbm, out_hbm, srcv, dstv, rows, zblk, acc, sem):
        c = lax.axis_index("c")
        s = lax.axis_index("s")
        row0 = jnp.minimum(s * _RPT, _RLAST)
        _zero_vmem(zblk, _ZR, w)
        for k in range(_RPT // _ZR):
            pltpu.sync_copy(zblk, acc.at[pl.ds(row0 + k * _ZR, _ZR)])
        plsc.subcore_barrier()
        base = (c * _NS + s) * _EPT

        def step(i, _):
            off = base + i * _CHUNK
            pltpu.sync_copy(src_hbm.at[pl.ds(off, _CHUNK)], srcv)
            pltpu.sync_copy(dst_hbm.at[pl.ds(off, _CHUNK)], dstv)
            pltpu.async_copy(q_hbm.at[srcv], rows, sem).wait()
            pltpu.sync_copy(rows, acc.at[dstv], add=True)
            return 0

        lax.fori_loop(0, _STEPS, step, 0)
        plsc.subcore_barrier()
        pltpu.sync_copy(acc.at[pl.ds(row0, _RPT)], out_hbm.at[c, pl.ds(row0, _RPT)])

    return agg


_sc_agg = {w: _make_sc_agg(w) for w in (128, 64, 32, 16)}


def _tc_first(degp_ref, x_ref, w_ref, dis_ref, q_ref):
    deg = degp_ref[0, :, 0:1] + degp_ref[1, :, 0:1] + 1.0
    dis = lax.rsqrt(deg)
    dis_ref[...] = dis
    q_ref[...] = jnp.dot(x_ref[...], w_ref[...], preferred_element_type=jnp.float32) * dis


def _tc_mid(part_ref, q_ref, dis_ref, b_ref, w_ref, o_ref):
    s = part_ref[0] + part_ref[1] + q_ref[...]
    h = jnp.maximum(s * dis_ref[...] + b_ref[...], 0.0)
    o_ref[...] = jnp.dot(h, w_ref[...], preferred_element_type=jnp.float32) * dis_ref[...]


def _tc_last(part_ref, q_ref, dis_ref, b_ref, o_ref):
    s = part_ref[0] + part_ref[1] + q_ref[...]
    o_ref[...] = jnp.maximum(s * dis_ref[...] + b_ref[...], 0.0)


def _row_spec(w):
    return pl.BlockSpec((_BN, w), lambda i: (i, 0))


def _part_spec(w):
    return pl.BlockSpec((_NC, _BN, w), lambda i: (0, i, 0))


def _full_spec(a, b):
    return pl.BlockSpec((a, b), lambda i: (0, 0))


def _tc_first_call(degp, x, W):
    return pl.pallas_call(
        _tc_first,
        grid=(_N // _BN,),
        in_specs=[_part_spec(_DEGW), _row_spec(128), _full_spec(128, 128)],
        out_specs=[_row_spec(1), _row_spec(128)],
        out_shape=[
            jax.ShapeDtypeStruct((_N, 1), jnp.float32),
            jax.ShapeDtypeStruct((_N, 128), jnp.float32),
        ],
    )(degp, x, W)


def _tc_mid_call(part, q, dis, b, W):
    w_in, w_out = W.shape
    return pl.pallas_call(
        _tc_mid,
        grid=(_N // _BN,),
        in_specs=[
            _part_spec(w_in),
            _row_spec(w_in),
            _row_spec(1),
            _full_spec(1, w_in),
            _full_spec(w_in, w_out),
        ],
        out_specs=_row_spec(w_out),
        out_shape=jax.ShapeDtypeStruct((_N, w_out), jnp.float32),
    )(part, q, dis, b.reshape(1, -1), W)


def _tc_last_call(part, q, dis, b):
    w = q.shape[1]
    return pl.pallas_call(
        _tc_last,
        grid=(_N // _BN,),
        in_specs=[_part_spec(w), _row_spec(w), _row_spec(1), _full_spec(1, w)],
        out_specs=_row_spec(w),
        out_shape=jax.ShapeDtypeStruct((_N, w), jnp.float32),
    )(part, q, dis, b.reshape(1, -1))


def kernel(x, edge_index, W1, b1, W3, b3, W4, b4, W2, b2):
    src = edge_index[0]
    dst = edge_index[1]

    degp = _sc_degree(dst)
    dis, q = _tc_first_call(degp, x, W1)

    for b_l, W_next in ((b1, W3), (b3, W4), (b4, W2)):
        part = _sc_agg[q.shape[1]](q, src, dst)
        q = _tc_mid_call(part, q, dis, b_l, W_next)

    part = _sc_agg[16](q, src, dst)
    return _tc_last_call(part, q, dis, b2)


# speedup vs baseline: 29.6692x; 2.5412x over previous
import functools

import jax
import jax.numpy as jnp
from jax import lax
from jax.experimental import pallas as pl
from jax.experimental.pallas import tpu as pltpu
from jax.experimental.pallas import tpu_sc as plsc

_N = 10000
_E = 320000
_NC = 2
_NS = 16
_NW = _NC * _NS
_EPT = _E // _NW
_CHUNK = 40
_STEPS = _EPT // _CHUNK
_RPT = 640
_RLAST = _N - _RPT
_ZR = 128
_DEGW = 16

_BN = 1000


def _zero_vmem(ref, rows, w):
    zero = jnp.zeros((16,), jnp.float32)

    def body(i, _):
        def inner(j, __):
            ref[i, pl.ds(j * 16, 16)] = zero
            return 0

        return lax.fori_loop(0, w // 16, inner, 0)

    lax.fori_loop(0, rows, body, 0)


def _fill_vmem(ref, rows, w, value):
    val = jnp.full((16,), value, jnp.float32)

    def body(i, _):
        def inner(j, __):
            ref[i, pl.ds(j * 16, 16)] = val
            return 0

        return lax.fori_loop(0, w // 16, inner, 0)

    lax.fori_loop(0, rows, body, 0)


_sc_mesh = plsc.VectorSubcoreMesh(core_axis_name="c", subcore_axis_name="s")


@functools.partial(
    pl.kernel,
    out_type=jax.ShapeDtypeStruct((_NC, _N, _DEGW), jnp.float32),
    mesh=_sc_mesh,
    scratch_types=[
        pltpu.VMEM((_E // _CHUNK // _NW, _CHUNK), jnp.int32),
        pltpu.VMEM((_CHUNK, _DEGW), jnp.float32),
        pltpu.VMEM((_ZR, _DEGW), jnp.float32),
        pltpu.VMEM_SHARED((_N, _DEGW), jnp.float32),
        pltpu.SemaphoreType.DMA,
    ],
    compiler_params=pltpu.CompilerParams(use_tc_tiling_on_sc=False),
)
def _sc_degree(dst_hbm, out_hbm, dst2, ones, zblk, acc, ssem):
    c = lax.axis_index("c")
    s = lax.axis_index("s")
    row0 = jnp.minimum(s * _RPT, _RLAST)
    tb = (c * _NS + s) * (_E // _CHUNK // _NW)
    pltpu.sync_copy(dst_hbm.at[pl.ds(tb, _E // _CHUNK // _NW)], dst2)
    _zero_vmem(zblk, _ZR, _DEGW)
    _fill_vmem(ones, _CHUNK, _DEGW, 1.0)
    for k in range(_RPT // _ZR):
        pltpu.sync_copy(zblk, acc.at[pl.ds(row0 + k * _ZR, _ZR)])
    plsc.subcore_barrier()

    for i in range(8):
        pltpu.async_copy(ones, acc.at[dst2.at[i]], ssem, add=True)

    def step(i, _):
        pltpu.make_async_copy(ones, acc.at[dst2.at[0]], ssem).wait()
        pltpu.async_copy(ones, acc.at[dst2.at[i]], ssem, add=True)
        return 0

    lax.fori_loop(8, _STEPS, step, 0)
    for _ in range(8):
        pltpu.make_async_copy(ones, acc.at[dst2.at[0]], ssem).wait()
    plsc.subcore_barrier()
    pltpu.sync_copy(acc.at[pl.ds(row0, _RPT)], out_hbm.at[c, pl.ds(row0, _RPT)])


_NBUF = 5


def _make_sc_agg(w):
    @functools.partial(
        pl.kernel,
        out_type=jax.ShapeDtypeStruct((_NC, _N, w), jnp.float32),
        mesh=_sc_mesh,
        scratch_types=[
            pltpu.VMEM((_STEPS, _CHUNK), jnp.int32),
            pltpu.VMEM((_STEPS, _CHUNK), jnp.int32),
            pltpu.VMEM((_NBUF, _CHUNK, w), jnp.float32),
            pltpu.VMEM_SHARED((_N, w), jnp.float32),
            pltpu.SemaphoreType.DMA,
            pltpu.SemaphoreType.DMA,
        ],
        compiler_params=pltpu.CompilerParams(use_tc_tiling_on_sc=False),
    )
    def agg(q_hbm, src_hbm, dst_hbm, out_hbm, src2, dst2, rows, acc, gsem, ssem):
        c = lax.axis_index("c")
        s = lax.axis_index("s")
        row0 = jnp.minimum(s * _RPT, _RLAST)
        tb = (c * _NS + s) * _STEPS
        pltpu.sync_copy(src_hbm.at[pl.ds(tb, _STEPS)], src2)
        pltpu.sync_copy(dst_hbm.at[pl.ds(tb, _STEPS)], dst2)

        zero = jnp.zeros((16,), jnp.float32)

        def zi(i, _):
            def zj(j, __):
                rows[0, i, pl.ds(j * 16, 16)] = zero
                return 0

            return lax.fori_loop(0, w // 16, zj, 0)

        lax.fori_loop(0, _CHUNK, zi, 0)
        for k in range(_RPT // _CHUNK):
            pltpu.sync_copy(rows.at[0], acc.at[pl.ds(row0 + k * _CHUNK, _CHUNK)])
        plsc.subcore_barrier()

        def g_issue(i):
            pltpu.async_copy(q_hbm.at[src2.at[i]], rows.at[lax.rem(i, _NBUF)], gsem)

        def g_wait_one():
            pltpu.make_async_copy(q_hbm.at[src2.at[0]], rows.at[0], gsem).wait()

        def s_issue(i):
            pltpu.async_copy(
                rows.at[lax.rem(i, _NBUF)], acc.at[dst2.at[i]], ssem, add=True
            )

        def s_wait_one():
            pltpu.make_async_copy(rows.at[0], acc.at[dst2.at[0]], ssem).wait()

        for i in range(3):
            g_issue(i)
        for i in (0, 1):
            g_wait_one()
            s_issue(i)
            g_issue(i + 3)

        def body(i, _):
            g_wait_one()
            s_issue(i)
            s_wait_one()
            g_issue(i + 3)
            return 0

        lax.fori_loop(2, _STEPS - 3, body, 0)
        for i in range(_STEPS - 3, _STEPS):
            g_wait_one()
            s_issue(i)
            s_wait_one()
        s_wait_one()
        s_wait_one()
        plsc.subcore_barrier()
        pltpu.sync_copy(acc.at[pl.ds(row0, _RPT)], out_hbm.at[c, pl.ds(row0, _RPT)])

    return agg


_sc_agg = {w: _make_sc_agg(w) for w in (128, 64, 32, 16)}


def _tc_first(degp_ref, x_ref, w_ref, dis_ref, q_ref):
    deg = degp_ref[0, :, 0:1] + degp_ref[1, :, 0:1] + 1.0
    dis = lax.rsqrt(deg)
    dis_ref[...] = dis
    q_ref[...] = jnp.dot(x_ref[...], w_ref[...], preferred_element_type=jnp.float32) * dis


def _tc_mid(part_ref, q_ref, dis_ref, b_ref, w_ref, o_ref):
    s = part_ref[0] + part_ref[1] + q_ref[...]
    h = jnp.maximum(s * dis_ref[...] + b_ref[...], 0.0)
    o_ref[...] = jnp.dot(h, w_ref[...], preferred_element_type=jnp.float32) * dis_ref[...]


def _tc_last(part_ref, q_ref, dis_ref, b_ref, o_ref):
    s = part_ref[0] + part_ref[1] + q_ref[...]
    o_ref[...] = jnp.maximum(s * dis_ref[...] + b_ref[...], 0.0)


def _row_spec(w):
    return pl.BlockSpec((_BN, w), lambda i: (i, 0))


def _part_spec(w):
    return pl.BlockSpec((_NC, _BN, w), lambda i: (0, i, 0))


def _full_spec(a, b):
    return pl.BlockSpec((a, b), lambda i: (0, 0))


def _tc_first_call(degp, x, W):
    return pl.pallas_call(
        _tc_first,
        grid=(_N // _BN,),
        in_specs=[_part_spec(_DEGW), _row_spec(128), _full_spec(128, 128)],
        out_specs=[_row_spec(1), _row_spec(128)],
        out_shape=[
            jax.ShapeDtypeStruct((_N, 1), jnp.float32),
            jax.ShapeDtypeStruct((_N, 128), jnp.float32),
        ],
    )(degp, x, W)


def _tc_mid_call(part, q, dis, b, W):
    w_in, w_out = W.shape
    return pl.pallas_call(
        _tc_mid,
        grid=(_N // _BN,),
        in_specs=[
            _part_spec(w_in),
            _row_spec(w_in),
            _row_spec(1),
            _full_spec(1, w_in),
            _full_spec(w_in, w_out),
        ],
        out_specs=_row_spec(w_out),
        out_shape=jax.ShapeDtypeStruct((_N, w_out), jnp.float32),
    )(part, q, dis, b.reshape(1, -1), W)


def _tc_last_call(part, q, dis, b):
    w = q.shape[1]
    return pl.pallas_call(
        _tc_last,
        grid=(_N // _BN,),
        in_specs=[_part_spec(w), _row_spec(w), _row_spec(1), _full_spec(1, w)],
        out_specs=_row_spec(w),
        out_shape=jax.ShapeDtypeStruct((_N, w), jnp.float32),
    )(part, q, dis, b.reshape(1, -1))


def kernel(x, edge_index, W1, b1, W3, b3, W4, b4, W2, b2):
    src2d = edge_index[0].reshape(_E // _CHUNK, _CHUNK)
    dst2d = edge_index[1].reshape(_E // _CHUNK, _CHUNK)

    degp = _sc_degree(dst2d)
    dis, q = _tc_first_call(degp, x, W1)

    for b_l, W_next in ((b1, W3), (b3, W4), (b4, W2)):
        part = _sc_agg[q.shape[1]](q, src2d, dst2d)
        q = _tc_mid_call(part, q, dis, b_l, W_next)

    part = _sc_agg[16](q, src2d, dst2d)
    return _tc_last_call(part, q, dis, b2)


# speedup vs baseline: 36.8837x; 1.2432x over previous
import functools

import jax
import jax.numpy as jnp
from jax import lax
from jax.experimental import pallas as pl
from jax.experimental.pallas import tpu as pltpu
from jax.experimental.pallas import tpu_sc as plsc

_N = 10000
_E = 320000
_NC = 2
_NS = 16
_NW = _NC * _NS
_EPT = _E // _NW
_CHUNK = 125
_STEPS = _EPT // _CHUNK
_RPT = 640
_RLAST = _N - _RPT
_ZR = 128
_DEGW = 16

_BN = 1000


def _zero_vmem(ref, rows, w):
    zero = jnp.zeros((16,), jnp.float32)

    def body(i, _):
        def inner(j, __):
            ref[i, pl.ds(j * 16, 16)] = zero
            return 0

        return lax.fori_loop(0, w // 16, inner, 0)

    lax.fori_loop(0, rows, body, 0)


def _fill_vmem(ref, rows, w, value):
    val = jnp.full((16,), value, jnp.float32)

    def body(i, _):
        def inner(j, __):
            ref[i, pl.ds(j * 16, 16)] = val
            return 0

        return lax.fori_loop(0, w // 16, inner, 0)

    lax.fori_loop(0, rows, body, 0)


_sc_mesh = plsc.VectorSubcoreMesh(core_axis_name="c", subcore_axis_name="s")


@functools.partial(
    pl.kernel,
    out_type=jax.ShapeDtypeStruct((_NC, _N, _DEGW), jnp.float32),
    mesh=_sc_mesh,
    scratch_types=[
        pltpu.VMEM((_E // _CHUNK // _NW, _CHUNK), jnp.int32),
        pltpu.VMEM((_CHUNK, _DEGW), jnp.float32),
        pltpu.VMEM((_ZR, _DEGW), jnp.float32),
        pltpu.VMEM_SHARED((_N, _DEGW), jnp.float32),
        pltpu.SemaphoreType.DMA,
    ],
    compiler_params=pltpu.CompilerParams(use_tc_tiling_on_sc=False),
)
def _sc_degree(dst_hbm, out_hbm, dst2, ones, zblk, acc, ssem):
    c = lax.axis_index("c")
    s = lax.axis_index("s")
    row0 = jnp.minimum(s * _RPT, _RLAST)
    tb = (c * _NS + s) * (_E // _CHUNK // _NW)
    pltpu.sync_copy(dst_hbm.at[pl.ds(tb, _E // _CHUNK // _NW)], dst2)
    _zero_vmem(zblk, _ZR, _DEGW)
    _fill_vmem(ones, _CHUNK, _DEGW, 1.0)
    for k in range(_RPT // _ZR):
        pltpu.sync_copy(zblk, acc.at[pl.ds(row0 + k * _ZR, _ZR)])
    plsc.subcore_barrier()

    for i in range(8):
        pltpu.async_copy(ones, acc.at[dst2.at[i]], ssem, add=True)

    def step(i, _):
        pltpu.make_async_copy(ones, acc.at[dst2.at[0]], ssem).wait()
        pltpu.async_copy(ones, acc.at[dst2.at[i]], ssem, add=True)
        return 0

    lax.fori_loop(8, _STEPS, step, 0)
    for _ in range(8):
        pltpu.make_async_copy(ones, acc.at[dst2.at[0]], ssem).wait()
    plsc.subcore_barrier()
    pltpu.sync_copy(acc.at[pl.ds(row0, _RPT)], out_hbm.at[c, pl.ds(row0, _RPT)])


_AGG_CFG = {128: (80, 3, 1), 64: (125, 5, 2), 32: (125, 5, 2), 16: (125, 5, 2)}


def _make_sc_agg(w):
    chunk, nbuf, lag = _AGG_CFG[w]
    steps = _EPT // chunk
    ahead = nbuf - lag

    @functools.partial(
        pl.kernel,
        out_type=jax.ShapeDtypeStruct((_NC, _N, w), jnp.float32),
        mesh=_sc_mesh,
        scratch_types=[
            pltpu.VMEM((steps, chunk), jnp.int32),
            pltpu.VMEM((steps, chunk), jnp.int32),
            pltpu.VMEM((nbuf, chunk, w), jnp.float32),
            pltpu.VMEM_SHARED((_N, w), jnp.float32),
            pltpu.SemaphoreType.DMA,
            pltpu.SemaphoreType.DMA,
        ],
        compiler_params=pltpu.CompilerParams(use_tc_tiling_on_sc=False),
    )
    def agg(q_hbm, src_hbm, dst_hbm, out_hbm, src2, dst2, rows, acc, gsem, ssem):
        c = lax.axis_index("c")
        s = lax.axis_index("s")
        row0 = jnp.minimum(s * _RPT, _RLAST)
        tb = (c * _NS + s) * steps
        pltpu.sync_copy(src_hbm.at[pl.ds(tb, steps)], src2)
        pltpu.sync_copy(dst_hbm.at[pl.ds(tb, steps)], dst2)

        zero = jnp.zeros((16,), jnp.float32)

        def zi(i, _):
            def zj(j, __):
                rows[0, i, pl.ds(j * 16, 16)] = zero
                return 0

            return lax.fori_loop(0, w // 16, zj, 0)

        lax.fori_loop(0, 80, zi, 0)
        for k in range(_RPT // 80):
            pltpu.sync_copy(
                rows.at[0, pl.ds(0, 80)], acc.at[pl.ds(row0 + k * 80, 80)]
            )
        plsc.subcore_barrier()

        def g_issue(i):
            pltpu.async_copy(q_hbm.at[src2.at[i]], rows.at[lax.rem(i, nbuf)], gsem)

        def g_wait_one():
            pltpu.make_async_copy(q_hbm.at[src2.at[0]], rows.at[0], gsem).wait()

        def s_issue(i):
            pltpu.async_copy(
                rows.at[lax.rem(i, nbuf)], acc.at[dst2.at[i]], ssem, add=True
            )

        def s_wait_one():
            pltpu.make_async_copy(rows.at[0], acc.at[dst2.at[0]], ssem).wait()

        for i in range(ahead):
            g_issue(i)
        for i in range(lag):
            g_wait_one()
            s_issue(i)
            g_issue(i + ahead)

        def body(i, _):
            g_wait_one()
            s_issue(i)
            s_wait_one()
            g_issue(i + ahead)
            return 0

        lax.fori_loop(lag, steps - ahead, body, 0)
        for i in range(steps - ahead, steps):
            g_wait_one()
            s_issue(i)
            s_wait_one()
        for _ in range(lag):
            s_wait_one()
        plsc.subcore_barrier()
        pltpu.sync_copy(acc.at[pl.ds(row0, _RPT)], out_hbm.at[c, pl.ds(row0, _RPT)])

    return agg


_sc_agg = {w: _make_sc_agg(w) for w in (128, 64, 32, 16)}


def _tc_first(degp_ref, x_ref, w_ref, dis_ref, q_ref):
    deg = degp_ref[0, :, 0:1] + degp_ref[1, :, 0:1] + 1.0
    dis = lax.rsqrt(deg)
    dis_ref[...] = dis
    q_ref[...] = jnp.dot(x_ref[...], w_ref[...], preferred_element_type=jnp.float32) * dis


def _tc_mid(part_ref, q_ref, dis_ref, b_ref, w_ref, o_ref):
    s = part_ref[0] + part_ref[1] + q_ref[...]
    h = jnp.maximum(s * dis_ref[...] + b_ref[...], 0.0)
    o_ref[...] = jnp.dot(h, w_ref[...], preferred_element_type=jnp.float32) * dis_ref[...]


def _tc_last(part_ref, q_ref, dis_ref, b_ref, o_ref):
    s = part_ref[0] + part_ref[1] + q_ref[...]
    o_ref[...] = jnp.maximum(s * dis_ref[...] + b_ref[...], 0.0)


def _row_spec(w):
    return pl.BlockSpec((_BN, w), lambda i: (i, 0))


def _part_spec(w):
    return pl.BlockSpec((_NC, _BN, w), lambda i: (0, i, 0))


def _full_spec(a, b):
    return pl.BlockSpec((a, b), lambda i: (0, 0))


def _tc_first_call(degp, x, W):
    return pl.pallas_call(
        _tc_first,
        grid=(_N // _BN,),
        in_specs=[_part_spec(_DEGW), _row_spec(128), _full_spec(128, 128)],
        out_specs=[_row_spec(1), _row_spec(128)],
        out_shape=[
            jax.ShapeDtypeStruct((_N, 1), jnp.float32),
            jax.ShapeDtypeStruct((_N, 128), jnp.float32),
        ],
    )(degp, x, W)


def _tc_mid_call(part, q, dis, b, W):
    w_in, w_out = W.shape
    return pl.pallas_call(
        _tc_mid,
        grid=(_N // _BN,),
        in_specs=[
            _part_spec(w_in),
            _row_spec(w_in),
            _row_spec(1),
            _full_spec(1, w_in),
            _full_spec(w_in, w_out),
        ],
        out_specs=_row_spec(w_out),
        out_shape=jax.ShapeDtypeStruct((_N, w_out), jnp.float32),
    )(part, q, dis, b.reshape(1, -1), W)


def _tc_last_call(part, q, dis, b):
    w = q.shape[1]
    return pl.pallas_call(
        _tc_last,
        grid=(_N // _BN,),
        in_specs=[_part_spec(w), _row_spec(w), _row_spec(1), _full_spec(1, w)],
        out_specs=_row_spec(w),
        out_shape=jax.ShapeDtypeStruct((_N, w), jnp.float32),
    )(part, q, dis, b.reshape(1, -1))


def kernel(x, edge_index, W1, b1, W3, b3, W4, b4, W2, b2):
    idx2d = {
        c: (
            edge_index[0].reshape(_E // c, c),
            edge_index[1].reshape(_E // c, c),
        )
        for c in {_CHUNK} | {cfg[0] for cfg in _AGG_CFG.values()}
    }

    degp = _sc_degree(idx2d[_CHUNK][1])
    dis, q = _tc_first_call(degp, x, W1)

    for b_l, W_next in ((b1, W3), (b3, W4), (b4, W2)):
        src2d, dst2d = idx2d[_AGG_CFG[q.shape[1]][0]]
        part = _sc_agg[q.shape[1]](q, src2d, dst2d)
        q = _tc_mid_call(part, q, dis, b_l, W_next)

    src2d, dst2d = idx2d[_AGG_CFG[16][0]]
    part = _sc_agg[16](q, src2d, dst2d)
    return _tc_last_call(part, q, dis, b2)
